# baseline (speedup 1.0000x reference)
import jax
import jax.numpy as jnp
from jax import lax
from jax.experimental import pallas as pl
from jax.experimental.pallas import tpu as pltpu
from jax.experimental.pallas import tpu_sc as plsc

B = 16384
NS = 5
D = 64
V = 1000000
L = 16
NC = 2
NSUB = 16
NW = NC * NSUB
NBLK = (V + 127) // 128
KB = 48
RB = 4
RND = 62
CAPI = RND * 128
CAPO = RND * 128
CHK = 2048
BPW = B // NW
CH = 128


def _k1_body(wtin_hbm, wtout_hbm, tgt_hbm, ctx_hbm,
             ein_hbm, eout_hbm, spin_hbm, spout_hbm,
             stage_v, cnt_v, bvoc_v, bdst_v, blk_v, rowbuf_v,
             didx_v, sval_v, semA, semB, semC):
    wid = lax.axis_index("s") * NC + lax.axis_index("c")
    bw0 = (NBLK * wid) // NW
    bw1 = (NBLK * (wid + 1)) // NW
    nloc = bw1 - bw0
    lane = lax.iota(jnp.int32, L)
    bw0v = jnp.full((L,), bw0, jnp.int32)
    bw1v = jnp.full((L,), bw1, jnp.int32)

    def reset_cnt():
        for i in range(272 // L):
            cnt_v[pl.ds(i * L, L)] = jnp.zeros((L,), jnp.int32)

    def scan_list(idx_hbm, n_idx):
        nch = n_idx // CHK

        def chunk_step(ch, carry):
            pltpu.sync_copy(idx_hbm.at[pl.ds(ch * CHK, CHK)], stage_v)

            def istep(i, c2):
                v = stage_v[pl.ds(i * L, L)]
                blk = jax.lax.shift_right_logical(v, 7)
                inr = (blk >= bw0v) & (blk < bw1v)
                loc = jnp.where(inr, blk - bw0v, 0)
                cnt1, lastm = plsc.scan_count(loc, mask=inr)
                base = plsc.load_gather(cnt_v, [loc])
                slot = loc * KB + jnp.minimum(base + cnt1 - 1, KB - 1)
                plsc.store_scatter(bvoc_v, [slot], v, mask=inr)
                pos = jnp.full((L,), ch * CHK + i * L, jnp.int32) + lane
                plsc.store_scatter(bdst_v, [slot], pos, mask=inr)
                newc = jnp.minimum(base + cnt1, KB)
                plsc.store_scatter(cnt_v, [loc], newc, mask=inr & lastm)
                return c2

            return lax.fori_loop(0, CHK // L, istep, carry)

        lax.fori_loop(0, nch, chunk_step, 0)

    def extract(wt_hbm, e_hbm, sp_hbm, cap, dumpbase):
        segbase = wid * cap
        lane7 = lane & 7
        lane3 = jax.lax.shift_right_logical(lane, 3)

        def round_tiles(r, parity):
            out = []
            for br in range(RB):
                bi = jnp.minimum(r * RB + br, nloc - 1)
                c = bw0 + bi
                for rt in range(8):
                    t = parity * (RB * 8) + br * 8 + rt
                    out.append((wt_hbm.at[pl.ds(rt * 8, 8),
                                          pl.ds(c * 128, 128)],
                                blk_v.at[t]))
            return out

        def issue_round(r, parity, sem_):
            for s_, d_ in round_tiles(r, parity):
                pltpu.async_copy(s_, d_, sem_)

        def wait_round(r, parity, sem_):
            for s_, d_ in round_tiles(r, parity):
                pltpu.make_async_copy(s_, d_, sem_).wait()

        def extract_round(r, parity):
            dumps = jnp.full((L,), dumpbase, jnp.int32) + lane
            for ii in range(128 // L):
                didx_v[pl.ds(ii * L, L)] = dumps + ii * L
            rcur = 0
            for br in range(RB):
                bi = r * RB + br
                nb = cnt_v[pl.ds(bi, L)][0]
                ngr = (nb + (L - 1)) // L

                def grp(g, rc0):
                    k0 = bi * KB + g * L
                    mvv = bvoc_v[pl.ds(k0, L)]
                    mdd = bdst_v[pl.ds(k0, L)]
                    nv = jnp.minimum(nb - g * L, L)
                    valid = lane < nv
                    rc = jnp.minimum(rc0, 128 - L)

                    def mstep(m, c3):
                        colsp = plsc.load_gather(
                            bvoc_v, [jnp.full((L,), k0 + m, jnp.int32)]) & 127
                        for k in range(D // L):
                            tvec = (parity * (RB * 8) + br * 8
                                    + k * 2 + lane3)
                            vals = plsc.load_gather(
                                blk_v, [tvec, lane7, colsp])
                            rowbuf_v[pl.ds((rc + m) * D + k * L, L)] = vals
                        return c3

                    lax.fori_loop(0, nv, mstep, 0)
                    didx_v[pl.ds(rc, L)] = jnp.where(valid, mdd, dumps)
                    sval_v[pl.ds(rc, L)] = jnp.full(
                        (L,), segbase + r * 128 + rc, jnp.int32) + lane
                    return rc0 + nv

                rcur = lax.fori_loop(0, ngr, grp, rcur)
            pltpu.sync_copy(
                rowbuf_v, e_hbm.at[pl.ds((segbase + r * 128) * D, 128 * D)])
            pltpu.async_copy(sval_v, sp_hbm.at[didx_v], semC).wait()

        issue_round(0, 0, semA)
        issue_round(1, 1, semB)

        def pair_body(i, carry):
            r0 = 2 * i
            r1 = 2 * i + 1
            wait_round(r0, 0, semA)
            extract_round(r0, 0)

            @pl.when(r0 + 2 < RND)
            def _():
                issue_round(r0 + 2, 0, semA)

            wait_round(r1, 1, semB)
            extract_round(r1, 1)

            @pl.when(r1 + 2 < RND)
            def _():
                issue_round(r1 + 2, 1, semB)

            return carry

        lax.fori_loop(0, RND // 2, pair_body, 0)

    reset_cnt()
    scan_list(tgt_hbm, B)
    extract(wtin_hbm, ein_hbm, spin_hbm, CAPI, B)
    reset_cnt()
    scan_list(ctx_hbm, B * NS)
    extract(wtout_hbm, eout_hbm, spout_hbm, CAPO, B * NS)


def _k2_body(ein_hbm, eout_hbm, spin_hbm, spout_hbm, out_hbm,
             tstage_v, cstage_v, tpair_v, cpair_v, hin_v, hout_v,
             vin_v, vout_v, outb_v, sem):
    wid = lax.axis_index("s") * NC + lax.axis_index("c")
    base = wid * BPW
    lane = lax.iota(jnp.int32, L)
    for c in range(BPW // CH):
        off = base + c * CH
        pltpu.sync_copy(spin_hbm.at[pl.ds(off, CH)], tstage_v)
        pltpu.sync_copy(spout_hbm.at[pl.ds(off * NS, CH * NS)], cstage_v)
        for i in range(CH // L):
            s = tstage_v[pl.ds(i * L, L)]
            tpair_v[pl.ds(i * L, L)] = jax.lax.shift_right_logical(s, 1)
            hin_v[pl.ds(i * L, L)] = s & 1
        for i in range(CH * NS // L):
            s = cstage_v[pl.ds(i * L, L)]
            cpair_v[pl.ds(i * L, L)] = jax.lax.shift_right_logical(s, 1)
            hout_v[pl.ds(i * L, L)] = s & 1
        pltpu.async_copy(ein_hbm.at[tpair_v], vin_v, sem).wait()
        for m in range(NS):
            pltpu.async_copy(eout_hbm.at[cpair_v.at[pl.ds(m * CH, CH)]],
                             vout_v.at[pl.ds(m * CH, CH)], sem).wait()
        for g in range(CH // L):
            row = jnp.full((L,), g * L, jnp.int32) + lane
            offin = hin_v[pl.ds(g * L, L)] * D
            rowj = [row * NS + j for j in range(NS)]
            offj = [plsc.load_gather(hout_v, [rowj[j]]) * D
                    for j in range(NS)]
            zero = jnp.zeros((L,), jnp.float32)

            def dstep(d, accs):
                dsp = jnp.full((L,), d, jnp.int32)
                vin_d = plsc.load_gather(vin_v, [row, dsp + offin])
                return tuple(
                    accs[j] + vin_d * plsc.load_gather(
                        vout_v, [rowj[j], dsp + offj[j]])
                    for j in range(NS))

            accs = lax.fori_loop(0, D, dstep, (zero,) * NS, unroll=4)
            for j in range(NS):
                plsc.store_scatter(outb_v, [rowj[j]], accs[j])
        pltpu.sync_copy(outb_v, out_hbm.at[pl.ds(off * NS, CH * NS)])


def _make_k1():
    mesh = plsc.VectorSubcoreMesh(core_axis_name="c", subcore_axis_name="s")
    return pl.kernel(
        _k1_body,
        mesh=mesh,
        compiler_params=pltpu.CompilerParams(needs_layout_passes=False,
                                             use_tc_tiling_on_sc=True),
        out_type=(
            jax.ShapeDtypeStruct((NW * CAPI * D,), jnp.float32),
            jax.ShapeDtypeStruct((NW * CAPO * D,), jnp.float32),
            jax.ShapeDtypeStruct((B + 128,), jnp.int32),
            jax.ShapeDtypeStruct((B * NS + 128,), jnp.int32),
        ),
        scratch_types=[
            pltpu.VMEM((CHK,), jnp.int32),
            pltpu.VMEM((272,), jnp.int32),
            pltpu.VMEM((256 * KB,), jnp.int32),
            pltpu.VMEM((256 * KB,), jnp.int32),
            pltpu.VMEM((2 * RB * 8, 8, 128), jnp.float32),
            pltpu.VMEM((128 * D,), jnp.float32),
            pltpu.VMEM((128,), jnp.int32),
            pltpu.VMEM((128,), jnp.int32),
            pltpu.SemaphoreType.DMA,
            pltpu.SemaphoreType.DMA,
            pltpu.SemaphoreType.DMA,
        ],
    )


def _make_k2():
    mesh = plsc.VectorSubcoreMesh(core_axis_name="c", subcore_axis_name="s")
    return pl.kernel(
        _k2_body,
        mesh=mesh,
        compiler_params=pltpu.CompilerParams(needs_layout_passes=False,
                                             use_tc_tiling_on_sc=False),
        out_type=jax.ShapeDtypeStruct((B * NS,), jnp.float32),
        scratch_types=[
            pltpu.VMEM((CH,), jnp.int32),
            pltpu.VMEM((CH * NS,), jnp.int32),
            pltpu.VMEM((CH,), jnp.int32),
            pltpu.VMEM((CH * NS,), jnp.int32),
            pltpu.VMEM((CH,), jnp.int32),
            pltpu.VMEM((CH * NS,), jnp.int32),
            pltpu.VMEM((CH, 2 * D), jnp.float32),
            pltpu.VMEM((CH * NS, 2 * D), jnp.float32),
            pltpu.VMEM((CH * NS,), jnp.float32),
            pltpu.SemaphoreType.DMA,
        ],
    )


def kernel(target, context, W_in, W_out):
    tgt = target.reshape(B).astype(jnp.int32)
    ctx = context.reshape(B * NS).astype(jnp.int32)
    wt_in = W_in.T
    wt_out = W_out.T
    e_in, e_out, sp_in, sp_out = _make_k1()(wt_in, wt_out, tgt, ctx)
    out = _make_k2()(e_in.reshape(NW * CAPI // 2, 2 * D),
                     e_out.reshape(NW * CAPO // 2, 2 * D),
                     sp_in, sp_out)
    return out.reshape(B, NS)


# speedup vs baseline: 23.6062x; 23.6062x over previous
import jax
import jax.numpy as jnp
from jax import lax
from jax.experimental import pallas as pl
from jax.experimental.pallas import tpu as pltpu
from jax.experimental.pallas import tpu_sc as plsc

B = 16384
NS = 5
D = 64
V = 1000000
L = 16
NC = 2
NSUB = 16
NW = NC * NSUB
BPW = B // NW
CH = 128
TCB = 512
NPBLK = (V + TCB - 1) // TCB
NP = NPBLK * (TCB // 2)


def _pack_body(wt_ref, p_ref):
    p_ref[:, pl.ds(0, D)] = wt_ref[:, pl.ds(0, TCB // 2)].T
    p_ref[:, pl.ds(D, D)] = wt_ref[:, pl.ds(TCB // 2, TCB // 2)].T


def _pack_table(wt):
    return pl.pallas_call(
        _pack_body,
        grid=(NPBLK,),
        in_specs=[pl.BlockSpec((D, TCB), lambda j: (0, j))],
        out_specs=pl.BlockSpec((TCB // 2, 2 * D), lambda j: (j, 0)),
        out_shape=jax.ShapeDtypeStruct((NP, 2 * D), jnp.float32),
    )(wt)


def _k2_body(pin_hbm, pout_hbm, tgt_hbm, ctx_hbm, out_hbm,
             tstage_v, cstage_v, tpair_v, cpair_v, hin_v, hout_v,
             vin_v, vout_v, outb_v, sem):
    wid = lax.axis_index("s") * NC + lax.axis_index("c")
    base = wid * BPW
    lane = lax.iota(jnp.int32, L)
    for c in range(BPW // CH):
        off = base + c * CH
        pltpu.sync_copy(tgt_hbm.at[pl.ds(off, CH)], tstage_v)
        pltpu.sync_copy(ctx_hbm.at[pl.ds(off * NS, CH * NS)], cstage_v)
        for i in range(CH // L):
            s = tstage_v[pl.ds(i * L, L)]
            tpair_v[pl.ds(i * L, L)] = (
                jax.lax.shift_left(jax.lax.shift_right_logical(s, 9), 8)
                + (s & 255))
            hin_v[pl.ds(i * L, L)] = jax.lax.shift_right_logical(s, 8) & 1
        for i in range(CH * NS // L):
            s = cstage_v[pl.ds(i * L, L)]
            cpair_v[pl.ds(i * L, L)] = (
                jax.lax.shift_left(jax.lax.shift_right_logical(s, 9), 8)
                + (s & 255))
            hout_v[pl.ds(i * L, L)] = jax.lax.shift_right_logical(s, 8) & 1
        pltpu.async_copy(pin_hbm.at[tpair_v], vin_v, sem).wait()
        for m in range(NS):
            pltpu.async_copy(pout_hbm.at[cpair_v.at[pl.ds(m * CH, CH)]],
                             vout_v.at[pl.ds(m * CH, CH)], sem).wait()
        for g in range(CH // L):
            row = jnp.full((L,), g * L, jnp.int32) + lane
            offin = hin_v[pl.ds(g * L, L)] * D
            rowj = [row * NS + j for j in range(NS)]
            offj = [plsc.load_gather(hout_v, [rowj[j]]) * D
                    for j in range(NS)]
            zero = jnp.zeros((L,), jnp.float32)

            def dstep(d, accs):
                dsp = jnp.full((L,), d, jnp.int32)
                vin_d = plsc.load_gather(vin_v, [row, dsp + offin])
                return tuple(
                    accs[j] + vin_d * plsc.load_gather(
                        vout_v, [rowj[j], dsp + offj[j]])
                    for j in range(NS))

            accs = lax.fori_loop(0, D, dstep, (zero,) * NS, unroll=4)
            for j in range(NS):
                plsc.store_scatter(outb_v, [rowj[j]], accs[j])
        pltpu.sync_copy(outb_v, out_hbm.at[pl.ds(off * NS, CH * NS)])


def _make_k2():
    mesh = plsc.VectorSubcoreMesh(core_axis_name="c", subcore_axis_name="s")
    return pl.kernel(
        _k2_body,
        mesh=mesh,
        compiler_params=pltpu.CompilerParams(needs_layout_passes=False,
                                             use_tc_tiling_on_sc=True),
        out_type=jax.ShapeDtypeStruct((B * NS,), jnp.float32),
        scratch_types=[
            pltpu.VMEM((CH,), jnp.int32),
            pltpu.VMEM((CH * NS,), jnp.int32),
            pltpu.VMEM((CH,), jnp.int32),
            pltpu.VMEM((CH * NS,), jnp.int32),
            pltpu.VMEM((CH,), jnp.int32),
            pltpu.VMEM((CH * NS,), jnp.int32),
            pltpu.VMEM((CH, 2 * D), jnp.float32),
            pltpu.VMEM((CH * NS, 2 * D), jnp.float32),
            pltpu.VMEM((CH * NS,), jnp.float32),
            pltpu.SemaphoreType.DMA,
        ],
    )


def kernel(target, context, W_in, W_out):
    tgt = target.reshape(B).astype(jnp.int32)
    ctx = context.reshape(B * NS).astype(jnp.int32)
    p_in = _pack_table(W_in.T)
    p_out = _pack_table(W_out.T)
    out = _make_k2()(p_in, p_out, tgt, ctx)
    return out.reshape(B, NS)


# speedup vs baseline: 37.8186x; 1.6021x over previous
import jax
import jax.numpy as jnp
from jax import lax
from jax.experimental import pallas as pl
from jax.experimental.pallas import tpu as pltpu
from jax.experimental.pallas import tpu_sc as plsc

B = 16384
NS = 5
D = 64
V = 1000000
L = 16
NC = 2
NSUB = 16
NW = NC * NSUB
BPW = B // NW
CH = 128
TCB = 1024
NPBLK = (V + TCB - 1) // TCB
NP = NPBLK * (TCB // 2)


def _pack_body(wt_ref, p_ref):
    eye = (jax.lax.broadcasted_iota(jnp.int32, (D, D), 0)
           == jax.lax.broadcasted_iota(jnp.int32, (D, D), 1)
           ).astype(jnp.float32)
    for h in range(2):
        x = wt_ref[:, pl.ds(h * (TCB // 2), TCB // 2)]
        p_ref[:, pl.ds(h * D, D)] = jax.lax.dot_general(
            x, eye, (((0,), (0,)), ((), ())),
            preferred_element_type=jnp.float32)


def _pack_table(wt):
    return pl.pallas_call(
        _pack_body,
        grid=(NPBLK,),
        in_specs=[pl.BlockSpec((D, TCB), lambda j: (0, j))],
        out_specs=pl.BlockSpec((TCB // 2, 2 * D), lambda j: (j, 0)),
        out_shape=jax.ShapeDtypeStruct((NP, 2 * D), jnp.float32),
    )(wt)


def _k2_body(pin_hbm, pout_hbm, tgt_hbm, ctx_hbm, out_hbm,
             tstage_v, cstage_v, tpair_v, cpair_v, hin_v, hout_v,
             vin_v, vout_v, outb_v, sem):
    wid = lax.axis_index("s") * NC + lax.axis_index("c")
    base = wid * BPW
    lane = lax.iota(jnp.int32, L)
    for c in range(BPW // CH):
        off = base + c * CH
        pltpu.sync_copy(tgt_hbm.at[pl.ds(off, CH)], tstage_v)
        pltpu.sync_copy(ctx_hbm.at[pl.ds(off * NS, CH * NS)], cstage_v)
        for i in range(CH // L):
            s = tstage_v[pl.ds(i * L, L)]
            tpair_v[pl.ds(i * L, L)] = (
                jax.lax.shift_left(jax.lax.shift_right_logical(s, 10), 9)
                + (s & 511))
            hin_v[pl.ds(i * L, L)] = jax.lax.shift_right_logical(s, 9) & 1
        for i in range(CH * NS // L):
            s = cstage_v[pl.ds(i * L, L)]
            cpair_v[pl.ds(i * L, L)] = (
                jax.lax.shift_left(jax.lax.shift_right_logical(s, 10), 9)
                + (s & 511))
            hout_v[pl.ds(i * L, L)] = jax.lax.shift_right_logical(s, 9) & 1
        pltpu.async_copy(pin_hbm.at[tpair_v], vin_v, sem).wait()
        for m in range(NS):
            pltpu.async_copy(pout_hbm.at[cpair_v.at[pl.ds(m * CH, CH)]],
                             vout_v.at[pl.ds(m * CH, CH)], sem).wait()
        for g in range(CH // L):
            row = jnp.full((L,), g * L, jnp.int32) + lane
            offin = hin_v[pl.ds(g * L, L)] * D
            rowj = [row * NS + j for j in range(NS)]
            offj = [plsc.load_gather(hout_v, [rowj[j]]) * D
                    for j in range(NS)]
            zero = jnp.zeros((L,), jnp.float32)

            def dstep(d, accs):
                dsp = jnp.full((L,), d, jnp.int32)
                vin_d = plsc.load_gather(vin_v, [row, dsp + offin])
                return tuple(
                    accs[j] + vin_d * plsc.load_gather(
                        vout_v, [rowj[j], dsp + offj[j]])
                    for j in range(NS))

            accs = lax.fori_loop(0, D, dstep, (zero,) * NS, unroll=4)
            for j in range(NS):
                plsc.store_scatter(outb_v, [rowj[j]], accs[j])
        pltpu.sync_copy(outb_v, out_hbm.at[pl.ds(off * NS, CH * NS)])


def _make_k2():
    mesh = plsc.VectorSubcoreMesh(core_axis_name="c", subcore_axis_name="s")
    return pl.kernel(
        _k2_body,
        mesh=mesh,
        compiler_params=pltpu.CompilerParams(needs_layout_passes=False,
                                             use_tc_tiling_on_sc=True),
        out_type=jax.ShapeDtypeStruct((B * NS,), jnp.float32),
        scratch_types=[
            pltpu.VMEM((CH,), jnp.int32),
            pltpu.VMEM((CH * NS,), jnp.int32),
            pltpu.VMEM((CH,), jnp.int32),
            pltpu.VMEM((CH * NS,), jnp.int32),
            pltpu.VMEM((CH,), jnp.int32),
            pltpu.VMEM((CH * NS,), jnp.int32),
            pltpu.VMEM((CH, 2 * D), jnp.float32),
            pltpu.VMEM((CH * NS, 2 * D), jnp.float32),
            pltpu.VMEM((CH * NS,), jnp.float32),
            pltpu.SemaphoreType.DMA,
        ],
    )


def kernel(target, context, W_in, W_out):
    tgt = target.reshape(B).astype(jnp.int32)
    ctx = context.reshape(B * NS).astype(jnp.int32)
    p_in = _pack_table(W_in.T)
    p_out = _pack_table(W_out.T)
    out = _make_k2()(p_in, p_out, tgt, ctx)
    return out.reshape(B, NS)


# speedup vs baseline: 73.9759x; 1.9561x over previous
import jax
import jax.numpy as jnp
from jax import lax
from jax.experimental import pallas as pl
from jax.experimental.pallas import tpu as pltpu
from jax.experimental.pallas import tpu_sc as plsc

B = 16384
NS = 5
D = 64
V = 1000000
L = 16
NC = 2
NSUB = 16
NW = NC * NSUB
BPW = B // NW
CH = 128
TCB = 4096
NPBLK = (V + TCB - 1) // TCB
NP = NPBLK * (TCB // 2)


def _pack_body(wt_ref, p_ref):
    eye = (jax.lax.broadcasted_iota(jnp.int32, (D, D), 0)
           == jax.lax.broadcasted_iota(jnp.int32, (D, D), 1)
           ).astype(jnp.float32)
    for h in range(2):
        x = wt_ref[:, pl.ds(h * (TCB // 2), TCB // 2)]
        p_ref[:, pl.ds(h * D, D)] = jax.lax.dot_general(
            x, eye, (((0,), (0,)), ((), ())),
            preferred_element_type=jnp.float32)


def _pack_table(wt):
    return pl.pallas_call(
        _pack_body,
        grid=(NPBLK,),
        in_specs=[pl.BlockSpec((D, TCB), lambda j: (0, j))],
        out_specs=pl.BlockSpec((TCB // 2, 2 * D), lambda j: (j, 0)),
        out_shape=jax.ShapeDtypeStruct((NP, 2 * D), jnp.float32),
    )(wt)


def _k2_body(pin_hbm, pout_hbm, tgt_hbm, ctx_hbm, out_hbm,
             tstage_v, cstage_v, tpair_v, cpair_v, hin_v, hout_v,
             vin_v, vout_v, outb_v, sem):
    wid = lax.axis_index("s") * NC + lax.axis_index("c")
    base = wid * BPW
    lane = lax.iota(jnp.int32, L)
    for c in range(BPW // CH):
        off = base + c * CH
        pltpu.sync_copy(tgt_hbm.at[pl.ds(off, CH)], tstage_v)
        pltpu.sync_copy(ctx_hbm.at[pl.ds(off * NS, CH * NS)], cstage_v)
        for i in range(CH // L):
            s = tstage_v[pl.ds(i * L, L)]
            tpair_v[pl.ds(i * L, L)] = (
                jax.lax.shift_left(jax.lax.shift_right_logical(s, 12), 11)
                + (s & 2047))
            hin_v[pl.ds(i * L, L)] = jax.lax.shift_right_logical(s, 11) & 1
        for i in range(CH * NS // L):
            s = cstage_v[pl.ds(i * L, L)]
            cpair_v[pl.ds(i * L, L)] = (
                jax.lax.shift_left(jax.lax.shift_right_logical(s, 12), 11)
                + (s & 2047))
            hout_v[pl.ds(i * L, L)] = jax.lax.shift_right_logical(s, 11) & 1
        pltpu.async_copy(pin_hbm.at[tpair_v], vin_v, sem).wait()
        for m in range(NS):
            pltpu.async_copy(pout_hbm.at[cpair_v.at[pl.ds(m * CH, CH)]],
                             vout_v.at[pl.ds(m * CH, CH)], sem).wait()
        for g in range(CH // L):
            row = jnp.full((L,), g * L, jnp.int32) + lane
            offin = hin_v[pl.ds(g * L, L)] * D
            rowj = [row * NS + j for j in range(NS)]
            offj = [plsc.load_gather(hout_v, [rowj[j]]) * D
                    for j in range(NS)]
            zero = jnp.zeros((L,), jnp.float32)

            def dstep(d, accs):
                dsp = jnp.full((L,), d, jnp.int32)
                vin_d = plsc.load_gather(vin_v, [row, dsp + offin])
                return tuple(
                    accs[j] + vin_d * plsc.load_gather(
                        vout_v, [rowj[j], dsp + offj[j]])
                    for j in range(NS))

            accs = lax.fori_loop(0, D, dstep, (zero,) * NS, unroll=4)
            for j in range(NS):
                plsc.store_scatter(outb_v, [rowj[j]], accs[j])
        pltpu.sync_copy(outb_v, out_hbm.at[pl.ds(off * NS, CH * NS)])


def _make_k2():
    mesh = plsc.VectorSubcoreMesh(core_axis_name="c", subcore_axis_name="s")
    return pl.kernel(
        _k2_body,
        mesh=mesh,
        compiler_params=pltpu.CompilerParams(needs_layout_passes=False,
                                             use_tc_tiling_on_sc=True),
        out_type=jax.ShapeDtypeStruct((B * NS,), jnp.float32),
        scratch_types=[
            pltpu.VMEM((CH,), jnp.int32),
            pltpu.VMEM((CH * NS,), jnp.int32),
            pltpu.VMEM((CH,), jnp.int32),
            pltpu.VMEM((CH * NS,), jnp.int32),
            pltpu.VMEM((CH,), jnp.int32),
            pltpu.VMEM((CH * NS,), jnp.int32),
            pltpu.VMEM((CH, 2 * D), jnp.float32),
            pltpu.VMEM((CH * NS, 2 * D), jnp.float32),
            pltpu.VMEM((CH * NS,), jnp.float32),
            pltpu.SemaphoreType.DMA,
        ],
    )


def kernel(target, context, W_in, W_out):
    tgt = target.reshape(B).astype(jnp.int32)
    ctx = context.reshape(B * NS).astype(jnp.int32)
    p_in = _pack_table(W_in.T)
    p_out = _pack_table(W_out.T)
    out = _make_k2()(p_in, p_out, tgt, ctx)
    return out.reshape(B, NS)


# speedup vs baseline: 88.4209x; 1.1953x over previous
import jax
import jax.numpy as jnp
from jax import lax
from jax.experimental import pallas as pl
from jax.experimental.pallas import tpu as pltpu
from jax.experimental.pallas import tpu_sc as plsc

B = 16384
NS = 5
D = 64
V = 1000000
L = 16
NC = 2
NSUB = 16
NW = NC * NSUB
BPW = B // NW
CH = 128
TCB = 8192
NPBLK = (V + TCB - 1) // TCB
NP = NPBLK * (TCB // 2)


def _pack_body(wt_ref, p_ref):
    eye = (jax.lax.broadcasted_iota(jnp.int32, (D, D), 0)
           == jax.lax.broadcasted_iota(jnp.int32, (D, D), 1)
           ).astype(jnp.float32)
    for h in range(2):
        x = wt_ref[:, pl.ds(h * (TCB // 2), TCB // 2)]
        p_ref[:, pl.ds(h * D, D)] = jax.lax.dot_general(
            x, eye, (((0,), (0,)), ((), ())),
            preferred_element_type=jnp.float32)


def _pack_table(wt):
    return pl.pallas_call(
        _pack_body,
        grid=(NPBLK,),
        in_specs=[pl.BlockSpec((D, TCB), lambda j: (0, j))],
        out_specs=pl.BlockSpec((TCB // 2, 2 * D), lambda j: (j, 0)),
        out_shape=jax.ShapeDtypeStruct((NP, 2 * D), jnp.float32),
    )(wt)


def _k2_body(pin_hbm, pout_hbm, tgt_hbm, ctx_hbm, out_hbm,
             tstage_v, cstage_v, tpair_v, cpair_v, hin_v, hout_v,
             vin_v, vout_v, outb_v, sem):
    wid = lax.axis_index("s") * NC + lax.axis_index("c")
    base = wid * BPW
    lane = lax.iota(jnp.int32, L)
    for c in range(BPW // CH):
        off = base + c * CH
        pltpu.sync_copy(tgt_hbm.at[pl.ds(off, CH)], tstage_v)
        pltpu.sync_copy(ctx_hbm.at[pl.ds(off * NS, CH * NS)], cstage_v)
        for i in range(CH // L):
            s = tstage_v[pl.ds(i * L, L)]
            tpair_v[pl.ds(i * L, L)] = (
                jax.lax.shift_left(jax.lax.shift_right_logical(s, 13), 12)
                + (s & 4095))
            hin_v[pl.ds(i * L, L)] = jax.lax.shift_right_logical(s, 12) & 1
        for i in range(CH * NS // L):
            s = cstage_v[pl.ds(i * L, L)]
            cpair_v[pl.ds(i * L, L)] = (
                jax.lax.shift_left(jax.lax.shift_right_logical(s, 13), 12)
                + (s & 4095))
            hout_v[pl.ds(i * L, L)] = jax.lax.shift_right_logical(s, 12) & 1
        pltpu.async_copy(pin_hbm.at[tpair_v], vin_v, sem).wait()
        for m in range(NS):
            pltpu.async_copy(pout_hbm.at[cpair_v.at[pl.ds(m * CH, CH)]],
                             vout_v.at[pl.ds(m * CH, CH)], sem).wait()
        for g in range(CH // L):
            row = jnp.full((L,), g * L, jnp.int32) + lane
            offin = hin_v[pl.ds(g * L, L)] * D
            rowj = [row * NS + j for j in range(NS)]
            offj = [plsc.load_gather(hout_v, [rowj[j]]) * D
                    for j in range(NS)]
            zero = jnp.zeros((L,), jnp.float32)

            def dstep(d, accs):
                dsp = jnp.full((L,), d, jnp.int32)
                vin_d = plsc.load_gather(vin_v, [row, dsp + offin])
                return tuple(
                    accs[j] + vin_d * plsc.load_gather(
                        vout_v, [rowj[j], dsp + offj[j]])
                    for j in range(NS))

            accs = lax.fori_loop(0, D, dstep, (zero,) * NS, unroll=4)
            for j in range(NS):
                plsc.store_scatter(outb_v, [rowj[j]], accs[j])
        pltpu.sync_copy(outb_v, out_hbm.at[pl.ds(off * NS, CH * NS)])


def _make_k2():
    mesh = plsc.VectorSubcoreMesh(core_axis_name="c", subcore_axis_name="s")
    return pl.kernel(
        _k2_body,
        mesh=mesh,
        compiler_params=pltpu.CompilerParams(needs_layout_passes=False,
                                             use_tc_tiling_on_sc=True),
        out_type=jax.ShapeDtypeStruct((B * NS,), jnp.float32),
        scratch_types=[
            pltpu.VMEM((CH,), jnp.int32),
            pltpu.VMEM((CH * NS,), jnp.int32),
            pltpu.VMEM((CH,), jnp.int32),
            pltpu.VMEM((CH * NS,), jnp.int32),
            pltpu.VMEM((CH,), jnp.int32),
            pltpu.VMEM((CH * NS,), jnp.int32),
            pltpu.VMEM((CH, 2 * D), jnp.float32),
            pltpu.VMEM((CH * NS, 2 * D), jnp.float32),
            pltpu.VMEM((CH * NS,), jnp.float32),
            pltpu.SemaphoreType.DMA,
        ],
    )


def kernel(target, context, W_in, W_out):
    tgt = target.reshape(B).astype(jnp.int32)
    ctx = context.reshape(B * NS).astype(jnp.int32)
    p_in = _pack_table(W_in.T)
    p_out = _pack_table(W_out.T)
    out = _make_k2()(p_in, p_out, tgt, ctx)
    return out.reshape(B, NS)


# speedup vs baseline: 97.8916x; 1.1071x over previous
import jax
import jax.numpy as jnp
from jax import lax
from jax.experimental import pallas as pl
from jax.experimental.pallas import tpu as pltpu
from jax.experimental.pallas import tpu_sc as plsc

B = 16384
NS = 5
D = 64
V = 1000000
L = 16
NC = 2
NSUB = 16
NW = NC * NSUB
BPW = B // NW
CH = 128
TCB = 16384
NPBLK = (V + TCB - 1) // TCB
NP = NPBLK * (TCB // 2)


def _pack_body(wt_ref, p_ref):
    eye = (jax.lax.broadcasted_iota(jnp.int32, (D, D), 0)
           == jax.lax.broadcasted_iota(jnp.int32, (D, D), 1)
           ).astype(jnp.float32)
    for h in range(2):
        x = wt_ref[:, pl.ds(h * (TCB // 2), TCB // 2)]
        p_ref[:, pl.ds(h * D, D)] = jax.lax.dot_general(
            x, eye, (((0,), (0,)), ((), ())),
            preferred_element_type=jnp.float32)


def _pack_table(wt):
    return pl.pallas_call(
        _pack_body,
        grid=(NPBLK,),
        in_specs=[pl.BlockSpec((D, TCB), lambda j: (0, j))],
        out_specs=pl.BlockSpec((TCB // 2, 2 * D), lambda j: (j, 0)),
        out_shape=jax.ShapeDtypeStruct((NP, 2 * D), jnp.float32),
    )(wt)


def _k2_body(pin_hbm, pout_hbm, tgt_hbm, ctx_hbm, out_hbm,
             tstage_v, cstage_v, tpair_v, cpair_v, hin_v, hout_v,
             vin_v, vout_v, outb_v, sem):
    wid = lax.axis_index("s") * NC + lax.axis_index("c")
    base = wid * BPW
    lane = lax.iota(jnp.int32, L)
    for c in range(BPW // CH):
        off = base + c * CH
        pltpu.sync_copy(tgt_hbm.at[pl.ds(off, CH)], tstage_v)
        pltpu.sync_copy(ctx_hbm.at[pl.ds(off * NS, CH * NS)], cstage_v)
        for i in range(CH // L):
            s = tstage_v[pl.ds(i * L, L)]
            tpair_v[pl.ds(i * L, L)] = (
                jax.lax.shift_left(jax.lax.shift_right_logical(s, 14), 13)
                + (s & 8191))
            hin_v[pl.ds(i * L, L)] = jax.lax.shift_right_logical(s, 13) & 1
        for i in range(CH * NS // L):
            s = cstage_v[pl.ds(i * L, L)]
            cpair_v[pl.ds(i * L, L)] = (
                jax.lax.shift_left(jax.lax.shift_right_logical(s, 14), 13)
                + (s & 8191))
            hout_v[pl.ds(i * L, L)] = jax.lax.shift_right_logical(s, 13) & 1
        pltpu.async_copy(pin_hbm.at[tpair_v], vin_v, sem).wait()
        for m in range(NS):
            pltpu.async_copy(pout_hbm.at[cpair_v.at[pl.ds(m * CH, CH)]],
                             vout_v.at[pl.ds(m * CH, CH)], sem).wait()
        for g in range(CH // L):
            row = jnp.full((L,), g * L, jnp.int32) + lane
            offin = hin_v[pl.ds(g * L, L)] * D
            rowj = [row * NS + j for j in range(NS)]
            offj = [plsc.load_gather(hout_v, [rowj[j]]) * D
                    for j in range(NS)]
            zero = jnp.zeros((L,), jnp.float32)

            def dstep(d, accs):
                dsp = jnp.full((L,), d, jnp.int32)
                vin_d = plsc.load_gather(vin_v, [row, dsp + offin])
                return tuple(
                    accs[j] + vin_d * plsc.load_gather(
                        vout_v, [rowj[j], dsp + offj[j]])
                    for j in range(NS))

            accs = lax.fori_loop(0, D, dstep, (zero,) * NS, unroll=4)
            for j in range(NS):
                plsc.store_scatter(outb_v, [rowj[j]], accs[j])
        pltpu.sync_copy(outb_v, out_hbm.at[pl.ds(off * NS, CH * NS)])


def _make_k2():
    mesh = plsc.VectorSubcoreMesh(core_axis_name="c", subcore_axis_name="s")
    return pl.kernel(
        _k2_body,
        mesh=mesh,
        compiler_params=pltpu.CompilerParams(needs_layout_passes=False,
                                             use_tc_tiling_on_sc=True),
        out_type=jax.ShapeDtypeStruct((B * NS,), jnp.float32),
        scratch_types=[
            pltpu.VMEM((CH,), jnp.int32),
            pltpu.VMEM((CH * NS,), jnp.int32),
            pltpu.VMEM((CH,), jnp.int32),
            pltpu.VMEM((CH * NS,), jnp.int32),
            pltpu.VMEM((CH,), jnp.int32),
            pltpu.VMEM((CH * NS,), jnp.int32),
            pltpu.VMEM((CH, 2 * D), jnp.float32),
            pltpu.VMEM((CH * NS, 2 * D), jnp.float32),
            pltpu.VMEM((CH * NS,), jnp.float32),
            pltpu.SemaphoreType.DMA,
        ],
    )


def kernel(target, context, W_in, W_out):
    tgt = target.reshape(B).astype(jnp.int32)
    ctx = context.reshape(B * NS).astype(jnp.int32)
    p_in = _pack_table(W_in.T)
    p_out = _pack_table(W_out.T)
    out = _make_k2()(p_in, p_out, tgt, ctx)
    return out.reshape(B, NS)
